# baseline (speedup 1.0000x reference)
import functools

import jax
import jax.numpy as jnp
from jax import lax
from jax.experimental import pallas as pl
from jax.experimental.pallas import tpu as pltpu
from jax.experimental.pallas import tpu_sc as plsc

_NC = 2
_NS = 16
_C = 96
_NBUF = 3
_F0 = 1.0 / 3.0
_LANES = 16


def _sc_edge_agg(x, x0, idxp, n_pad, nch0, nch1):
    n, d = x.shape
    kd = d // _LANES
    rpt = (n // _NS) // 8 * 8
    last = n - (_NS - 1) * rpt
    ng0 = nch0 // _NBUF
    ng1 = nch1 // _NBUF
    assert ng0 % 2 == 1 and ng1 % 2 == 1 and min(ng0, ng1) >= 3

    mesh = plsc.VectorSubcoreMesh(
        core_axis_name="c", subcore_axis_name="s",
        num_cores=_NC, num_subcores=_NS)

    @functools.partial(
        pl.kernel,
        out_type=jax.ShapeDtypeStruct((_NC, n, d), jnp.float32),
        mesh=mesh,
        scratch_types=(
            [
                pltpu.VMEM_SHARED((n_pad, d), jnp.float32),
                pltpu.VMEM((_NBUF, _C, d), jnp.float32),
                pltpu.VMEM((2, _NBUF, 3, _C), jnp.int32),
            ]
            + [pltpu.SemaphoreType.DMA] * (3 * _NBUF + 2)
        ),
    )
    def body(x_hbm, x0_hbm, idx_hbm, out_hbm, acc_sh, xb, ib, *sems):
        sem_a = sems[0:_NBUF]
        sem_b = sems[_NBUF:2 * _NBUF]
        sem_c = sems[2 * _NBUF:3 * _NBUF]
        sem_i = sems[3 * _NBUF:]
        c = lax.axis_index("c")
        s = lax.axis_index("s")
        ng = jnp.where(c == 0, ng0, ng1)
        chunk0 = jnp.where(c == 0, s * nch0, _NS * nch0 + s * nch1)

        def zrow(r, carry):
            for k in range(kd):
                xb[0, r, pl.ds(k * _LANES, _LANES)] = jnp.zeros(
                    (_LANES,), jnp.float32)
            return carry
        lax.fori_loop(0, _C, zrow, 0)
        zrows = n_pad // _NS
        base = s * zrows
        for k in range(zrows // _C):
            pltpu.sync_copy(xb.at[0], acc_sh.at[pl.ds(base + k * _C, _C)])
        zrem = zrows - (zrows // _C) * _C
        if zrem:
            pltpu.sync_copy(xb.at[0, pl.ds(0, zrem)],
                            acc_sh.at[pl.ds(base + zrows - zrem, zrem)])
        plsc.subcore_barrier()

        def relu_buf(b):
            def row(r, rc):
                for k in range(kd):
                    sl = pl.ds(k * _LANES, _LANES)
                    xb[b, r, sl] = jnp.maximum(xb[b, r, sl], 0.0)
                return rc
            lax.fori_loop(0, _C, row, 0)

        def load_idx_group(g, q):
            return pltpu.async_copy(
                idx_hbm.at[pl.ds(chunk0 + g * _NBUF, _NBUF)], ib.at[q],
                sem_i[q])

        def wait_idx(q):
            pltpu.make_async_copy(
                idx_hbm.at[pl.ds(0, _NBUF)], ib.at[q], sem_i[q]).wait()

        def drain_scatter(b):
            pltpu.make_async_copy(
                xb.at[b], acc_sh.at[pl.ds(0, _C)], sem_c[b]).wait()

        def do_group(g, q, drain):
            @pl.when(g + 1 < ng)
            def _():
                load_idx_group(g + 1, 1 - q)
            gx = []
            for b in range(_NBUF):
                if drain:
                    drain_scatter(b)
                gx.append(pltpu.async_copy(
                    x_hbm.at[ib.at[q, b, 0]], xb.at[b], sem_a[b]))
            ga = []
            for b in range(_NBUF):
                gx[b].wait()
                ga.append(pltpu.async_copy(
                    x0_hbm.at[ib.at[q, b, 2]], xb.at[b], sem_b[b], add=True))
            for b in range(_NBUF):
                ga[b].wait()
                relu_buf(b)
                pltpu.async_copy(
                    xb.at[b], acc_sh.at[ib.at[q, b, 1]], sem_c[b], add=True)

        load_idx_group(0, 0).wait()
        do_group(0, 0, drain=False)

        def pair(i, carry):
            g = 1 + 2 * i
            wait_idx(1)
            do_group(g, 1, drain=True)
            wait_idx(0)
            do_group(g + 1, 0, drain=True)
            return carry
        lax.fori_loop(0, (ng - 1) // 2, pair, 0)

        for b in range(_NBUF):
            drain_scatter(b)

        plsc.subcore_barrier()

        @pl.when(s < _NS - 1)
        def _():
            pltpu.sync_copy(acc_sh.at[pl.ds(s * rpt, rpt)],
                            out_hbm.at[c, pl.ds(s * rpt, rpt)])

        @pl.when(s == _NS - 1)
        def _():
            pltpu.sync_copy(acc_sh.at[pl.ds((_NS - 1) * rpt, last)],
                            out_hbm.at[c, pl.ds((_NS - 1) * rpt, last)])

    return body(x, x0, idxp)


def _mlp_stage1(x, aggs, w1, b1, eps):
    n, d = x.shape
    h = w1.shape[1]
    blk = 2000
    nblk = n // blk

    def body(x_ref, agg_ref, w1_ref, b1_ref, eps_ref, y_ref):
        hblk = ((1.0 + eps_ref[0, 0]) * x_ref[...]
                + agg_ref[0] + agg_ref[1])
        y_ref[...] = jnp.dot(hblk, w1_ref[...],
                             preferred_element_type=jnp.float32) + b1_ref[...]

    return pl.pallas_call(
        body,
        grid=(nblk,),
        in_specs=[
            pl.BlockSpec((blk, d), lambda i: (i, 0)),
            pl.BlockSpec((_NC, blk, d), lambda i: (0, i, 0)),
            pl.BlockSpec((d, h), lambda i: (0, 0)),
            pl.BlockSpec((1, h), lambda i: (0, 0)),
            pl.BlockSpec(memory_space=pltpu.SMEM),
        ],
        out_specs=pl.BlockSpec((blk, h), lambda i: (i, 0)),
        out_shape=jax.ShapeDtypeStruct((n, h), jnp.float32),
    )(x, aggs, w1, b1, eps)


def _mlp_stage2(y, g1, be1, w2, b2, g2, be2):
    def body(y_ref, g1_ref, be1_ref, w2_ref, b2_ref, g2_ref, be2_ref, o_ref):
        y = y_ref[...]
        m1 = jnp.mean(y, axis=0, keepdims=True)
        v1 = jnp.mean((y - m1) ** 2, axis=0, keepdims=True)
        y = g1_ref[...] * (y - m1) / jnp.sqrt(v1 + 1e-5) + be1_ref[...]
        y = jnp.maximum(y, 0.0)
        z = jnp.dot(y, w2_ref[...],
                    preferred_element_type=jnp.float32) + b2_ref[...]
        m2 = jnp.mean(z, axis=0, keepdims=True)
        v2 = jnp.mean((z - m2) ** 2, axis=0, keepdims=True)
        z = g2_ref[...] * (z - m2) / jnp.sqrt(v2 + 1e-5) + be2_ref[...]
        o_ref[...] = jnp.maximum(z, 0.0)

    n, h = y.shape
    return pl.pallas_call(
        body,
        out_shape=jax.ShapeDtypeStruct((n, h), jnp.float32),
    )(y, g1, be1, w2, b2, g2, be2)


def kernel(x, edge_index, x0, bridge_index, W1, b1, g1, be1, W2, b2, g2, be2,
           eps):
    n, d = x.shape
    e = bridge_index.shape[0]
    h = W1.shape[1]
    nw = _NC * _NS

    quantum = _NS * _C * _NBUF

    def _odd(v):
        v = max(3, v)
        return v if v % 2 == 1 else v + 1

    ng0 = _odd(int(round(e * _F0 / quantum)))
    e0 = ng0 * quantum
    ng1 = _odd(-(-(e - e0) // quantum))
    nch0, nch1 = ng0 * _NBUF, ng1 * _NBUF
    pad = e0 + ng1 * quantum - e
    src = edge_index[0]
    dst = edge_index[1]
    if pad:
        zpad = jnp.zeros((pad,), jnp.int32)
        src = jnp.concatenate([src, zpad])
        dst = jnp.concatenate([dst, jnp.full((pad,), n, jnp.int32)])
        bridge_index = jnp.concatenate([bridge_index, zpad])
    ntot = _NS * (nch0 + nch1)
    idxp = jnp.stack(
        [src.reshape(ntot, _C), dst.reshape(ntot, _C),
         bridge_index.reshape(ntot, _C)], axis=1)

    n_pad = -(-(n + 1) // (_NS * 8)) * (_NS * 8)

    aggs = _sc_edge_agg(x, x0, idxp, n_pad, nch0, nch1)

    y = _mlp_stage1(x, aggs, W1, b1.reshape(1, h), eps.reshape(1, 1))
    return _mlp_stage2(y, g1.reshape(1, h), be1.reshape(1, h), W2,
                       b2.reshape(1, h), g2.reshape(1, h), be2.reshape(1, h))


# speedup vs baseline: 1.0887x; 1.0887x over previous
import functools

import jax
import jax.numpy as jnp
from jax import lax
from jax.experimental import pallas as pl
from jax.experimental.pallas import tpu as pltpu
from jax.experimental.pallas import tpu_sc as plsc

_NC = 2
_NS = 16
_C = 96
_NBUF = 3
_F0 = 0.5
_LANES = 16


def _sc_edge_agg(x, x0, idxp, n_pad, nch0, nch1):
    n, d = x.shape
    kd = d // _LANES
    rpt = (n // _NS) // 8 * 8
    last = n - (_NS - 1) * rpt
    ng0 = nch0 // _NBUF
    ng1 = nch1 // _NBUF
    assert ng0 % 2 == 1 and ng1 % 2 == 1 and min(ng0, ng1) >= 3

    mesh = plsc.VectorSubcoreMesh(
        core_axis_name="c", subcore_axis_name="s",
        num_cores=_NC, num_subcores=_NS)

    @functools.partial(
        pl.kernel,
        out_type=jax.ShapeDtypeStruct((_NC, n, d), jnp.float32),
        mesh=mesh,
        scratch_types=(
            [
                pltpu.VMEM_SHARED((n_pad, d), jnp.float32),
                pltpu.VMEM((_NBUF, _C, d), jnp.float32),
                pltpu.VMEM((2, _NBUF, 3, _C), jnp.int32),
            ]
            + [pltpu.SemaphoreType.DMA] * (3 * _NBUF + 2)
        ),
    )
    def body(x_hbm, x0_hbm, idx_hbm, out_hbm, acc_sh, xb, ib, *sems):
        sem_a = sems[0:_NBUF]
        sem_b = sems[_NBUF:2 * _NBUF]
        sem_c = sems[2 * _NBUF:3 * _NBUF]
        sem_i = sems[3 * _NBUF:]
        c = lax.axis_index("c")
        s = lax.axis_index("s")
        ng = jnp.where(c == 0, ng0, ng1)
        chunk0 = jnp.where(c == 0, s * nch0, _NS * nch0 + s * nch1)

        def zrow(r, carry):
            for k in range(kd):
                xb[0, r, pl.ds(k * _LANES, _LANES)] = jnp.zeros(
                    (_LANES,), jnp.float32)
            return carry
        lax.fori_loop(0, _C, zrow, 0)
        zrows = n_pad // _NS
        base = s * zrows
        for k in range(zrows // _C):
            pltpu.sync_copy(xb.at[0], acc_sh.at[pl.ds(base + k * _C, _C)])
        zrem = zrows - (zrows // _C) * _C
        if zrem:
            pltpu.sync_copy(xb.at[0, pl.ds(0, zrem)],
                            acc_sh.at[pl.ds(base + zrows - zrem, zrem)])
        plsc.subcore_barrier()

        def relu_buf(b):
            def row(r, rc):
                for k in range(kd):
                    sl = pl.ds(k * _LANES, _LANES)
                    xb[b, r, sl] = jnp.maximum(xb[b, r, sl], 0.0)
                return rc
            lax.fori_loop(0, _C, row, 0)

        def load_idx_group(g, q):
            return pltpu.async_copy(
                idx_hbm.at[pl.ds(chunk0 + g * _NBUF, _NBUF)], ib.at[q],
                sem_i[q])

        def wait_idx(q):
            pltpu.make_async_copy(
                idx_hbm.at[pl.ds(0, _NBUF)], ib.at[q], sem_i[q]).wait()

        def drain_scatter(b):
            pltpu.make_async_copy(
                xb.at[b], acc_sh.at[pl.ds(0, _C)], sem_c[b]).wait()

        def do_group(g, q, drain):
            @pl.when(g + 1 < ng)
            def _():
                load_idx_group(g + 1, 1 - q)
            gx = []
            for b in range(_NBUF):
                if drain:
                    drain_scatter(b)
                gx.append(pltpu.async_copy(
                    x_hbm.at[ib.at[q, b, 0]], xb.at[b], sem_a[b]))
            ga = []
            for b in range(_NBUF):
                gx[b].wait()
                ga.append(pltpu.async_copy(
                    x0_hbm.at[ib.at[q, b, 2]], xb.at[b], sem_b[b], add=True))
            for b in range(_NBUF):
                ga[b].wait()
                relu_buf(b)
                pltpu.async_copy(
                    xb.at[b], acc_sh.at[ib.at[q, b, 1]], sem_c[b], add=True)

        load_idx_group(0, 0).wait()
        do_group(0, 0, drain=False)

        def pair(i, carry):
            g = 1 + 2 * i
            wait_idx(1)
            do_group(g, 1, drain=True)
            wait_idx(0)
            do_group(g + 1, 0, drain=True)
            return carry
        lax.fori_loop(0, (ng - 1) // 2, pair, 0)

        for b in range(_NBUF):
            drain_scatter(b)

        plsc.subcore_barrier()

        @pl.when(s < _NS - 1)
        def _():
            pltpu.sync_copy(acc_sh.at[pl.ds(s * rpt, rpt)],
                            out_hbm.at[c, pl.ds(s * rpt, rpt)])

        @pl.when(s == _NS - 1)
        def _():
            pltpu.sync_copy(acc_sh.at[pl.ds((_NS - 1) * rpt, last)],
                            out_hbm.at[c, pl.ds((_NS - 1) * rpt, last)])

    return body(x, x0, idxp)


def _mlp_stage1(x, aggs, w1, b1, eps):
    n, d = x.shape
    h = w1.shape[1]
    blk = 2000
    nblk = n // blk

    def body(x_ref, agg_ref, w1_ref, b1_ref, eps_ref, y_ref):
        hblk = ((1.0 + eps_ref[0, 0]) * x_ref[...]
                + agg_ref[0] + agg_ref[1])
        y_ref[...] = jnp.dot(hblk, w1_ref[...],
                             preferred_element_type=jnp.float32) + b1_ref[...]

    return pl.pallas_call(
        body,
        grid=(nblk,),
        in_specs=[
            pl.BlockSpec((blk, d), lambda i: (i, 0)),
            pl.BlockSpec((_NC, blk, d), lambda i: (0, i, 0)),
            pl.BlockSpec((d, h), lambda i: (0, 0)),
            pl.BlockSpec((1, h), lambda i: (0, 0)),
            pl.BlockSpec(memory_space=pltpu.SMEM),
        ],
        out_specs=pl.BlockSpec((blk, h), lambda i: (i, 0)),
        out_shape=jax.ShapeDtypeStruct((n, h), jnp.float32),
    )(x, aggs, w1, b1, eps)


def _mlp_stage2(y, g1, be1, w2, b2, g2, be2):
    def body(y_ref, g1_ref, be1_ref, w2_ref, b2_ref, g2_ref, be2_ref, o_ref):
        y = y_ref[...]
        m1 = jnp.mean(y, axis=0, keepdims=True)
        v1 = jnp.mean((y - m1) ** 2, axis=0, keepdims=True)
        y = g1_ref[...] * (y - m1) / jnp.sqrt(v1 + 1e-5) + be1_ref[...]
        y = jnp.maximum(y, 0.0)
        z = jnp.dot(y, w2_ref[...],
                    preferred_element_type=jnp.float32) + b2_ref[...]
        m2 = jnp.mean(z, axis=0, keepdims=True)
        v2 = jnp.mean((z - m2) ** 2, axis=0, keepdims=True)
        z = g2_ref[...] * (z - m2) / jnp.sqrt(v2 + 1e-5) + be2_ref[...]
        o_ref[...] = jnp.maximum(z, 0.0)

    n, h = y.shape
    return pl.pallas_call(
        body,
        out_shape=jax.ShapeDtypeStruct((n, h), jnp.float32),
    )(y, g1, be1, w2, b2, g2, be2)


def kernel(x, edge_index, x0, bridge_index, W1, b1, g1, be1, W2, b2, g2, be2,
           eps):
    n, d = x.shape
    e = bridge_index.shape[0]
    h = W1.shape[1]
    nw = _NC * _NS

    quantum = _NS * _C * _NBUF

    def _odd(v):
        v = max(3, v)
        return v if v % 2 == 1 else v + 1

    ng0 = _odd(int(round(e * _F0 / quantum)))
    e0 = ng0 * quantum
    ng1 = _odd(-(-(e - e0) // quantum))
    nch0, nch1 = ng0 * _NBUF, ng1 * _NBUF
    pad = e0 + ng1 * quantum - e
    n_pad = -(-(n + 1) // (_NS * 8)) * (_NS * 8)
    src = edge_index[0]
    dst = edge_index[1]
    if pad:
        zpad = jnp.zeros((pad,), jnp.int32)
        src = jnp.concatenate([src, zpad])
        dummy = n + jnp.arange(pad, dtype=jnp.int32) % max(n_pad - n, 1)
        dst = jnp.concatenate([dst, dummy])
        bridge_index = jnp.concatenate([bridge_index, zpad])
    ntot = _NS * (nch0 + nch1)
    idxp = jnp.stack(
        [src.reshape(ntot, _C), dst.reshape(ntot, _C),
         bridge_index.reshape(ntot, _C)], axis=1)

    aggs = _sc_edge_agg(x, x0, idxp, n_pad, nch0, nch1)

    y = _mlp_stage1(x, aggs, W1, b1.reshape(1, h), eps.reshape(1, 1))
    return _mlp_stage2(y, g1.reshape(1, h), be1.reshape(1, h), W2,
                       b2.reshape(1, h), g2.reshape(1, h), be2.reshape(1, h))


# speedup vs baseline: 1.2036x; 1.1056x over previous
import functools

import jax
import jax.numpy as jnp
from jax import lax
from jax.experimental import pallas as pl
from jax.experimental.pallas import tpu as pltpu
from jax.experimental.pallas import tpu_sc as plsc

_NC = 2
_NS = 16
_C = 96
_NBUF = 3
_F0 = 2.0 / 3.0
_LANES = 16


def _sc_edge_agg(x, x0, idxp, n_pad, nch0, nch1):
    n, d = x.shape
    kd = d // _LANES
    rpt = (n // _NS) // 8 * 8
    last = n - (_NS - 1) * rpt
    ng0 = nch0 // _NBUF
    ng1 = nch1 // _NBUF
    assert ng0 % 2 == 1 and ng1 % 2 == 1 and min(ng0, ng1) >= 3

    mesh = plsc.VectorSubcoreMesh(
        core_axis_name="c", subcore_axis_name="s",
        num_cores=_NC, num_subcores=_NS)

    @functools.partial(
        pl.kernel,
        out_type=jax.ShapeDtypeStruct((_NC, n, d), jnp.float32),
        mesh=mesh,
        scratch_types=(
            [
                pltpu.VMEM_SHARED((n_pad, d), jnp.float32),
                pltpu.VMEM((_NBUF, _C, d), jnp.float32),
                pltpu.VMEM((2, _NBUF, 3, _C), jnp.int32),
            ]
            + [pltpu.SemaphoreType.DMA] * (3 * _NBUF + 2)
        ),
    )
    def body(x_hbm, x0_hbm, idx_hbm, out_hbm, acc_sh, xb, ib, *sems):
        sem_a = sems[0:_NBUF]
        sem_b = sems[_NBUF:2 * _NBUF]
        sem_c = sems[2 * _NBUF:3 * _NBUF]
        sem_i = sems[3 * _NBUF:]
        c = lax.axis_index("c")
        s = lax.axis_index("s")
        ng = jnp.where(c == 0, ng0, ng1)
        chunk0 = jnp.where(c == 0, s * nch0, _NS * nch0 + s * nch1)

        def zrow(r, carry):
            for k in range(kd):
                xb[0, r, pl.ds(k * _LANES, _LANES)] = jnp.zeros(
                    (_LANES,), jnp.float32)
            return carry
        lax.fori_loop(0, _C, zrow, 0)
        zrows = n_pad // _NS
        base = s * zrows
        for k in range(zrows // _C):
            pltpu.sync_copy(xb.at[0], acc_sh.at[pl.ds(base + k * _C, _C)])
        zrem = zrows - (zrows // _C) * _C
        if zrem:
            pltpu.sync_copy(xb.at[0, pl.ds(0, zrem)],
                            acc_sh.at[pl.ds(base + zrows - zrem, zrem)])
        plsc.subcore_barrier()

        def relu_buf(b):
            def row(r, rc):
                for k in range(kd):
                    sl = pl.ds(k * _LANES, _LANES)
                    xb[b, r, sl] = jnp.maximum(xb[b, r, sl], 0.0)
                return rc
            lax.fori_loop(0, _C, row, 0)

        def load_idx_group(g, q):
            return pltpu.async_copy(
                idx_hbm.at[pl.ds(chunk0 + g * _NBUF, _NBUF)], ib.at[q],
                sem_i[q])

        def wait_idx(q):
            pltpu.make_async_copy(
                idx_hbm.at[pl.ds(0, _NBUF)], ib.at[q], sem_i[q]).wait()

        def drain_scatter(b):
            pltpu.make_async_copy(
                xb.at[b], acc_sh.at[pl.ds(0, _C)], sem_c[b]).wait()

        def do_group(g, q, drain):
            @pl.when(g + 1 < ng)
            def _():
                load_idx_group(g + 1, 1 - q)
            gx = []
            for b in range(_NBUF):
                if drain:
                    drain_scatter(b)
                gx.append(pltpu.async_copy(
                    x_hbm.at[ib.at[q, b, 0]], xb.at[b], sem_a[b]))
            ga = []
            for b in range(_NBUF):
                gx[b].wait()
                ga.append(pltpu.async_copy(
                    x0_hbm.at[ib.at[q, b, 2]], xb.at[b], sem_b[b], add=True))
            for b in range(_NBUF):
                ga[b].wait()
                relu_buf(b)
                pltpu.async_copy(
                    xb.at[b], acc_sh.at[ib.at[q, b, 1]], sem_c[b], add=True)

        load_idx_group(0, 0).wait()
        do_group(0, 0, drain=False)

        def pair(i, carry):
            g = 1 + 2 * i
            wait_idx(1)
            do_group(g, 1, drain=True)
            wait_idx(0)
            do_group(g + 1, 0, drain=True)
            return carry
        lax.fori_loop(0, (ng - 1) // 2, pair, 0)

        for b in range(_NBUF):
            drain_scatter(b)

        plsc.subcore_barrier()

        @pl.when(s < _NS - 1)
        def _():
            pltpu.sync_copy(acc_sh.at[pl.ds(s * rpt, rpt)],
                            out_hbm.at[c, pl.ds(s * rpt, rpt)])

        @pl.when(s == _NS - 1)
        def _():
            pltpu.sync_copy(acc_sh.at[pl.ds((_NS - 1) * rpt, last)],
                            out_hbm.at[c, pl.ds((_NS - 1) * rpt, last)])

    return body(x, x0, idxp)


def _mlp_stage1(x, aggs, w1, b1, eps):
    n, d = x.shape
    h = w1.shape[1]
    blk = 2000
    nblk = n // blk

    def body(x_ref, agg_ref, w1_ref, b1_ref, eps_ref, y_ref):
        hblk = ((1.0 + eps_ref[0, 0]) * x_ref[...]
                + agg_ref[0] + agg_ref[1])
        y_ref[...] = jnp.dot(hblk, w1_ref[...],
                             preferred_element_type=jnp.float32) + b1_ref[...]

    return pl.pallas_call(
        body,
        grid=(nblk,),
        in_specs=[
            pl.BlockSpec((blk, d), lambda i: (i, 0)),
            pl.BlockSpec((_NC, blk, d), lambda i: (0, i, 0)),
            pl.BlockSpec((d, h), lambda i: (0, 0)),
            pl.BlockSpec((1, h), lambda i: (0, 0)),
            pl.BlockSpec(memory_space=pltpu.SMEM),
        ],
        out_specs=pl.BlockSpec((blk, h), lambda i: (i, 0)),
        out_shape=jax.ShapeDtypeStruct((n, h), jnp.float32),
    )(x, aggs, w1, b1, eps)


def _mlp_stage2(y, g1, be1, w2, b2, g2, be2):
    def body(y_ref, g1_ref, be1_ref, w2_ref, b2_ref, g2_ref, be2_ref, o_ref):
        y = y_ref[...]
        m1 = jnp.mean(y, axis=0, keepdims=True)
        v1 = jnp.mean((y - m1) ** 2, axis=0, keepdims=True)
        y = g1_ref[...] * (y - m1) / jnp.sqrt(v1 + 1e-5) + be1_ref[...]
        y = jnp.maximum(y, 0.0)
        z = jnp.dot(y, w2_ref[...],
                    preferred_element_type=jnp.float32) + b2_ref[...]
        m2 = jnp.mean(z, axis=0, keepdims=True)
        v2 = jnp.mean((z - m2) ** 2, axis=0, keepdims=True)
        z = g2_ref[...] * (z - m2) / jnp.sqrt(v2 + 1e-5) + be2_ref[...]
        o_ref[...] = jnp.maximum(z, 0.0)

    n, h = y.shape
    return pl.pallas_call(
        body,
        out_shape=jax.ShapeDtypeStruct((n, h), jnp.float32),
    )(y, g1, be1, w2, b2, g2, be2)


def kernel(x, edge_index, x0, bridge_index, W1, b1, g1, be1, W2, b2, g2, be2,
           eps):
    n, d = x.shape
    e = bridge_index.shape[0]
    h = W1.shape[1]
    nw = _NC * _NS

    quantum = _NS * _C * _NBUF

    def _odd(v):
        v = max(3, v)
        return v if v % 2 == 1 else v + 1

    ng0 = _odd(int(round(e * _F0 / quantum)))
    e0 = ng0 * quantum
    ng1 = _odd(-(-(e - e0) // quantum))
    nch0, nch1 = ng0 * _NBUF, ng1 * _NBUF
    pad = e0 + ng1 * quantum - e
    n_pad = -(-(n + 1) // (_NS * 8)) * (_NS * 8)
    src = edge_index[0]
    dst = edge_index[1]
    if pad:
        zpad = jnp.zeros((pad,), jnp.int32)
        src = jnp.concatenate([src, zpad])
        dummy = n + jnp.arange(pad, dtype=jnp.int32) % max(n_pad - n, 1)
        dst = jnp.concatenate([dst, dummy])
        bridge_index = jnp.concatenate([bridge_index, zpad])
    ntot = _NS * (nch0 + nch1)
    idxp = jnp.stack(
        [src.reshape(ntot, _C), dst.reshape(ntot, _C),
         bridge_index.reshape(ntot, _C)], axis=1)

    aggs = _sc_edge_agg(x, x0, idxp, n_pad, nch0, nch1)

    y = _mlp_stage1(x, aggs, W1, b1.reshape(1, h), eps.reshape(1, 1))
    return _mlp_stage2(y, g1.reshape(1, h), be1.reshape(1, h), W2,
                       b2.reshape(1, h), g2.reshape(1, h), be2.reshape(1, h))
